# baseline (speedup 1.0000x reference)
import functools

import jax
import jax.numpy as jnp
from jax import lax
from jax.experimental import pallas as pl
from jax.experimental.pallas import tpu as pltpu
from jax.experimental.pallas import tpu_sc as plsc

N = 10000
E = 320000
D = 128
K = 64
INF = 3 * D
DP = D // 2

NC = 2
NS = 16
NW = NC * NS
PER_W = E // NW
C = 128
NFULL = PER_W // C
TAIL = PER_W - NFULL * C
N_ACC = 10240
ROWS_PER_SUB = N_ACC // NS

_sc_mesh = plsc.VectorSubcoreMesh(core_axis_name="c", subcore_axis_name="s")


def _worker_id():
    return lax.axis_index("s") * NC + lax.axis_index("c")


@functools.partial(
    pl.kernel,
    out_type=(
        jax.ShapeDtypeStruct((E, DP), jnp.int32),
        jax.ShapeDtypeStruct((E, DP), jnp.int32),
    ),
    mesh=_sc_mesh,
    scratch_types=(
        pltpu.VMEM((3, C), jnp.int32),
        pltpu.VMEM((3, C), jnp.int32),
        pltpu.VMEM((3, C, DP), jnp.int32),
        pltpu.VMEM((3, C, DP), jnp.int32),
        pltpu.SemaphoreType.DMA,
        pltpu.SemaphoreType.DMA,
        pltpu.SemaphoreType.DMA,
        pltpu.SemaphoreType.DMA,
        pltpu.SemaphoreType.DMA,
        pltpu.SemaphoreType.DMA,
        pltpu.SemaphoreType.DMA,
        pltpu.SemaphoreType.DMA,
        pltpu.SemaphoreType.DMA,
        pltpu.VMEM((TAIL,), jnp.int32),
        pltpu.VMEM((TAIL,), jnp.int32),
        pltpu.VMEM((TAIL, DP), jnp.int32),
        pltpu.VMEM((TAIL, DP), jnp.int32),
    ),
    compiler_params=pltpu.CompilerParams(use_tc_tiling_on_sc=False),
)
def _gather(x_hbm, src_hbm, tgt_hbm, ni_hbm, nj_hbm,
            idx_s, idx_t, rows_s, rows_t,
            semI0, semI1, semI2, semG0, semG1, semG2, semW0, semW1, semW2,
            idx_s3, idx_t3, rows_s3, rows_t3):
    semI = (semI0, semI1, semI2)
    semG = (semG0, semG1, semG2)
    semW = (semW0, semW1, semW2)
    wid = _worker_id()
    base_w = wid * PER_W

    def fire_idx(j, b):
        base = base_w + j * C
        pltpu.async_copy(src_hbm.at[pl.ds(base, C)], idx_s.at[b], semI[b])
        pltpu.async_copy(tgt_hbm.at[pl.ds(base, C)], idx_t.at[b], semI[b])

    def fire_gather(j, b):
        base = base_w + j * C
        pltpu.make_async_copy(src_hbm.at[pl.ds(base, C)], idx_s.at[b],
                              semI[b]).wait()
        pltpu.make_async_copy(tgt_hbm.at[pl.ds(base, C)], idx_t.at[b],
                              semI[b]).wait()
        pltpu.async_copy(x_hbm.at[idx_s.at[b]], rows_s.at[b], semG[b])
        pltpu.async_copy(x_hbm.at[idx_t.at[b]], rows_t.at[b], semG[b])

    def fire_write(j, b):
        base = base_w + j * C
        pltpu.make_async_copy(x_hbm.at[idx_s.at[b]], rows_s.at[b],
                              semG[b]).wait()
        pltpu.make_async_copy(x_hbm.at[idx_t.at[b]], rows_t.at[b],
                              semG[b]).wait()
        pltpu.async_copy(rows_s.at[b], ni_hbm.at[pl.ds(base, C)], semW[b])
        pltpu.async_copy(rows_t.at[b], nj_hbm.at[pl.ds(base, C)], semW[b])

    def wait_write(j, b):
        base = base_w + j * C
        pltpu.make_async_copy(rows_s.at[b], ni_hbm.at[pl.ds(base, C)],
                              semW[b]).wait()
        pltpu.make_async_copy(rows_t.at[b], nj_hbm.at[pl.ds(base, C)],
                              semW[b]).wait()

    fire_idx(0, 0)
    fire_idx(1, 1)
    fire_gather(0, 0)

    def g_loop(g, _):
        for t in range(3):
            j = 3 * g + t
            bA = (t + 2) % 3

            if t == 0:
                @pl.when(g > 0)
                def _():
                    wait_write(j - 1, bA)
                fire_idx(j + 2, bA)
            else:
                wait_write(j - 1, bA)

                @pl.when(g < (NFULL // 3) - 1)
                def _():
                    fire_idx(j + 2, bA)

            if t < 2:
                fire_gather(j + 1, (t + 1) % 3)
            else:
                @pl.when(g < (NFULL // 3) - 1)
                def _():
                    fire_gather(j + 1, (t + 1) % 3)

            fire_write(j, t)
        return 0

    lax.fori_loop(0, NFULL // 3, g_loop, 0)
    wait_write(NFULL - 1, (NFULL - 1) % 3)

    base = base_w + NFULL * C
    pltpu.sync_copy(src_hbm.at[pl.ds(base, TAIL)], idx_s3)
    pltpu.sync_copy(tgt_hbm.at[pl.ds(base, TAIL)], idx_t3)
    a = pltpu.async_copy(x_hbm.at[idx_s3], rows_s3, semG0)
    b = pltpu.async_copy(x_hbm.at[idx_t3], rows_t3, semG1)
    a.wait()
    b.wait()
    pltpu.sync_copy(rows_s3, ni_hbm.at[pl.ds(base, TAIL)])
    pltpu.sync_copy(rows_t3, nj_hbm.at[pl.ds(base, TAIL)])


@functools.partial(
    pl.kernel,
    out_type=jax.ShapeDtypeStruct((NC, N_ACC, D), jnp.float32),
    mesh=_sc_mesh,
    scratch_types=(
        pltpu.VMEM_SHARED((N_ACC, D), jnp.float32),
        pltpu.VMEM((2, C), jnp.int32),
        pltpu.VMEM((2, C, D), jnp.float32),
        pltpu.SemaphoreType.DMA,
        pltpu.SemaphoreType.DMA,
        pltpu.VMEM((TAIL,), jnp.int32),
        pltpu.VMEM((TAIL, D), jnp.float32),
    ),
)
def _scatter(z_hbm, src_hbm, zero_hbm, part_hbm,
             acc, idx_v, rows_v, sem0, sem1, idx_v3, rows_v3):
    cid = lax.axis_index("c")
    sid = lax.axis_index("s")
    wid = _worker_id()
    base_w = wid * PER_W

    pltpu.sync_copy(zero_hbm.at[pl.ds(sid * ROWS_PER_SUB, ROWS_PER_SUB)],
                    acc.at[pl.ds(sid * ROWS_PER_SUB, ROWS_PER_SUB)])
    plsc.subcore_barrier()

    def fire(j, b, sem):
        base = base_w + j * C
        pltpu.async_copy(src_hbm.at[pl.ds(base, C)], idx_v.at[b], sem)
        pltpu.async_copy(z_hbm.at[pl.ds(base, C)], rows_v.at[b], sem)

    def drain_add(j, b, sem):
        base = base_w + j * C
        pltpu.make_async_copy(src_hbm.at[pl.ds(base, C)], idx_v.at[b], sem).wait()
        pltpu.make_async_copy(z_hbm.at[pl.ds(base, C)], rows_v.at[b], sem).wait()
        pltpu.sync_copy(rows_v.at[b], acc.at[idx_v.at[b]], add=True)

    fire(0, 0, sem0)

    def g_loop(g, _):
        fire(2 * g + 1, 1, sem1)
        drain_add(2 * g, 0, sem0)

        @pl.when(2 * g + 2 < NFULL)
        def _():
            fire(2 * g + 2, 0, sem0)

        drain_add(2 * g + 1, 1, sem1)
        return 0

    lax.fori_loop(0, NFULL // 2, g_loop, 0)

    base = base_w + NFULL * C
    pltpu.sync_copy(src_hbm.at[pl.ds(base, TAIL)], idx_v3)
    pltpu.sync_copy(z_hbm.at[pl.ds(base, TAIL)], rows_v3)
    pltpu.sync_copy(rows_v3, acc.at[idx_v3], add=True)

    plsc.subcore_barrier()
    pltpu.sync_copy(acc.at[pl.ds(sid * ROWS_PER_SUB, ROWS_PER_SUB)],
                    part_hbm.at[cid].at[pl.ds(sid * ROWS_PER_SUB, ROWS_PER_SUB)])


BE = 2000


def _edge_mlp_body(rij_ref, cut_ref, cs_ref, pw_ref, ni_ref, nj_ref,
                   w1_ref, w2_ref, w2g_ref, wg_ref, wm_ref, z_ref):
    ni = ni_ref[...]
    nj = nj_ref[...]
    r = rij_ref[...]
    rb = (1.0 / r).astype(jnp.bfloat16)
    delta = (ni - nj) * rb
    fe = jnp.concatenate([ni, nj, delta], axis=1)
    g = jnp.dot(fe, wg_ref[...], preferred_element_type=jnp.float32)
    g = 0.5 * jnp.tanh(0.5 * g) + 0.5
    m = jnp.dot(fe, wm_ref[...], preferred_element_type=jnp.float32)
    m = jnp.where(m > 0, m, jnp.exp(jnp.minimum(m, 0.0)) - 1.0)
    pw = pw_ref[...]
    gate = jnp.dot(pw.astype(jnp.bfloat16), w2g_ref[...],
                   preferred_element_type=jnp.float32)
    gate = 0.5 * jnp.tanh(0.5 * gate) + 0.5
    z2 = jnp.dot((pw * gate).astype(jnp.bfloat16), w2_ref[...],
                 preferred_element_type=jnp.float32)
    z1 = jnp.dot(cs_ref[...].astype(jnp.bfloat16), w1_ref[...],
                 preferred_element_type=jnp.float32)
    mask = (r < cut_ref[0]).astype(jnp.float32)
    z_ref[...] = g * m * (z1 + z2) * mask


def _edge_mlp(rij2, cutoff, cs, pw, ni, nj, w1, w2, w2g, wg, wm):
    grid = (E // BE,)
    full = lambda shape: pl.BlockSpec(shape, lambda i: (0,) * len(shape))
    return pl.pallas_call(
        _edge_mlp_body,
        grid=grid,
        in_specs=[
            pl.BlockSpec((BE, 1), lambda i: (i, 0)),
            pl.BlockSpec(memory_space=pltpu.SMEM),
            pl.BlockSpec((BE, K), lambda i: (i, 0)),
            pl.BlockSpec((BE, K), lambda i: (i, 0)),
            pl.BlockSpec((BE, D), lambda i: (i, 0)),
            pl.BlockSpec((BE, D), lambda i: (i, 0)),
            full((K, D)),
            full((K, D)),
            full((K, K)),
            full((INF, D)),
            full((INF, D)),
        ],
        out_specs=pl.BlockSpec((BE, D), lambda i: (i, 0)),
        out_shape=jax.ShapeDtypeStruct((E, D), jnp.float32),
    )(rij2, cutoff, cs, pw, ni, nj, w1, w2, w2g, wg, wm)


BN = 1000


def _combine_body(x_ref, p_ref, o_ref):
    o_ref[...] = x_ref[...] + p_ref[0] + p_ref[1]


def _combine(x, parts):
    return pl.pallas_call(
        _combine_body,
        grid=(N // BN,),
        in_specs=[
            pl.BlockSpec((BN, D), lambda i: (i, 0)),
            pl.BlockSpec((NC, BN, D), lambda i: (0, i, 0)),
        ],
        out_specs=pl.BlockSpec((BN, D), lambda i: (i, 0)),
        out_shape=jax.ShapeDtypeStruct((N, D), jnp.float32),
    )(x, parts)


def kernel(input, nodes, edge_sources, edge_targets, rij, combine_sets,
           plane_wave, cutoff, W1, W2, W2g, Wg, Wm):
    bf = jnp.bfloat16
    x32 = jax.lax.bitcast_convert_type(
        input.astype(bf).reshape(N, DP, 2), jnp.int32)
    ni32, nj32 = _gather(x32, edge_sources, edge_targets)
    ni = jax.lax.bitcast_convert_type(ni32, bf).reshape(E, D)
    nj = jax.lax.bitcast_convert_type(nj32, bf).reshape(E, D)
    z = _edge_mlp(rij[:, None], cutoff, combine_sets, plane_wave, ni, nj,
                  W1.astype(bf), W2.astype(bf), W2g.astype(bf),
                  Wg.astype(bf), Wm.astype(bf))
    zero = jnp.zeros((N_ACC, D), jnp.float32)
    parts = _scatter(z, edge_sources, zero)
    return _combine(input, parts)


# speedup vs baseline: 2.2183x; 2.2183x over previous
import functools

import jax
import jax.numpy as jnp
from jax import lax
from jax.experimental import pallas as pl
from jax.experimental.pallas import tpu as pltpu
from jax.experimental.pallas import tpu_sc as plsc

N = 10000
E = 320000
D = 128
K = 64
INF = 3 * D
DP = D // 2

NC = 2
NS = 16
NW = NC * NS
PER_W = E // NW
C = 128
NFULL = PER_W // C
TAIL = PER_W - NFULL * C
N_ACC = 10240
ROWS_PER_SUB = N_ACC // NS

_sc_mesh = plsc.VectorSubcoreMesh(core_axis_name="c", subcore_axis_name="s")


def _worker_id():
    return lax.axis_index("s") * NC + lax.axis_index("c")


@functools.partial(
    pl.kernel,
    out_type=(
        jax.ShapeDtypeStruct((E, D), jnp.float32),
        jax.ShapeDtypeStruct((E, D), jnp.float32),
    ),
    mesh=_sc_mesh,
    scratch_types=(
        pltpu.VMEM((3, C), jnp.int32),
        pltpu.VMEM((3, C), jnp.int32),
        pltpu.VMEM((3, C, D), jnp.float32),
        pltpu.VMEM((3, C, D), jnp.float32),
        pltpu.SemaphoreType.DMA,
        pltpu.SemaphoreType.DMA,
        pltpu.SemaphoreType.DMA,
        pltpu.SemaphoreType.DMA,
        pltpu.SemaphoreType.DMA,
        pltpu.SemaphoreType.DMA,
        pltpu.SemaphoreType.DMA,
        pltpu.SemaphoreType.DMA,
        pltpu.SemaphoreType.DMA,
        pltpu.VMEM((TAIL,), jnp.int32),
        pltpu.VMEM((TAIL,), jnp.int32),
        pltpu.VMEM((TAIL, D), jnp.float32),
        pltpu.VMEM((TAIL, D), jnp.float32),
    ),
)
def _gather(x_hbm, src_hbm, tgt_hbm, ni_hbm, nj_hbm,
            idx_s, idx_t, rows_s, rows_t,
            semI0, semI1, semI2, semG0, semG1, semG2, semW0, semW1, semW2,
            idx_s3, idx_t3, rows_s3, rows_t3):
    semI = (semI0, semI1, semI2)
    semG = (semG0, semG1, semG2)
    semW = (semW0, semW1, semW2)
    wid = _worker_id()
    base_w = wid * PER_W

    def fire_idx(j, b):
        base = base_w + j * C
        pltpu.async_copy(src_hbm.at[pl.ds(base, C)], idx_s.at[b], semI[b])
        pltpu.async_copy(tgt_hbm.at[pl.ds(base, C)], idx_t.at[b], semI[b])

    def fire_gather(j, b):
        base = base_w + j * C
        pltpu.make_async_copy(src_hbm.at[pl.ds(base, C)], idx_s.at[b],
                              semI[b]).wait()
        pltpu.make_async_copy(tgt_hbm.at[pl.ds(base, C)], idx_t.at[b],
                              semI[b]).wait()
        pltpu.async_copy(x_hbm.at[idx_s.at[b]], rows_s.at[b], semG[b])
        pltpu.async_copy(x_hbm.at[idx_t.at[b]], rows_t.at[b], semG[b])

    def fire_write(j, b):
        base = base_w + j * C
        pltpu.make_async_copy(x_hbm.at[idx_s.at[b]], rows_s.at[b],
                              semG[b]).wait()
        pltpu.make_async_copy(x_hbm.at[idx_t.at[b]], rows_t.at[b],
                              semG[b]).wait()
        pltpu.async_copy(rows_s.at[b], ni_hbm.at[pl.ds(base, C)], semW[b])
        pltpu.async_copy(rows_t.at[b], nj_hbm.at[pl.ds(base, C)], semW[b])

    def wait_write(j, b):
        base = base_w + j * C
        pltpu.make_async_copy(rows_s.at[b], ni_hbm.at[pl.ds(base, C)],
                              semW[b]).wait()
        pltpu.make_async_copy(rows_t.at[b], nj_hbm.at[pl.ds(base, C)],
                              semW[b]).wait()

    fire_idx(0, 0)
    fire_idx(1, 1)
    fire_gather(0, 0)

    def g_loop(g, _):
        for t in range(3):
            j = 3 * g + t
            bA = (t + 2) % 3

            if t == 0:
                @pl.when(g > 0)
                def _():
                    wait_write(j - 1, bA)
                fire_idx(j + 2, bA)
            else:
                wait_write(j - 1, bA)

                @pl.when(g < (NFULL // 3) - 1)
                def _():
                    fire_idx(j + 2, bA)

            if t < 2:
                fire_gather(j + 1, (t + 1) % 3)
            else:
                @pl.when(g < (NFULL // 3) - 1)
                def _():
                    fire_gather(j + 1, (t + 1) % 3)

            fire_write(j, t)
        return 0

    lax.fori_loop(0, NFULL // 3, g_loop, 0)
    wait_write(NFULL - 1, (NFULL - 1) % 3)

    base = base_w + NFULL * C
    pltpu.sync_copy(src_hbm.at[pl.ds(base, TAIL)], idx_s3)
    pltpu.sync_copy(tgt_hbm.at[pl.ds(base, TAIL)], idx_t3)
    a = pltpu.async_copy(x_hbm.at[idx_s3], rows_s3, semG0)
    b = pltpu.async_copy(x_hbm.at[idx_t3], rows_t3, semG1)
    a.wait()
    b.wait()
    pltpu.sync_copy(rows_s3, ni_hbm.at[pl.ds(base, TAIL)])
    pltpu.sync_copy(rows_t3, nj_hbm.at[pl.ds(base, TAIL)])


@functools.partial(
    pl.kernel,
    out_type=jax.ShapeDtypeStruct((NC, N_ACC, D), jnp.float32),
    mesh=_sc_mesh,
    scratch_types=(
        pltpu.VMEM_SHARED((N_ACC, D), jnp.float32),
        pltpu.VMEM((2, C), jnp.int32),
        pltpu.VMEM((2, C, D), jnp.float32),
        pltpu.SemaphoreType.DMA,
        pltpu.SemaphoreType.DMA,
        pltpu.VMEM((TAIL,), jnp.int32),
        pltpu.VMEM((TAIL, D), jnp.float32),
    ),
)
def _scatter(z_hbm, src_hbm, zero_hbm, part_hbm,
             acc, idx_v, rows_v, sem0, sem1, idx_v3, rows_v3):
    cid = lax.axis_index("c")
    sid = lax.axis_index("s")
    wid = _worker_id()
    base_w = wid * PER_W

    pltpu.sync_copy(zero_hbm.at[pl.ds(sid * ROWS_PER_SUB, ROWS_PER_SUB)],
                    acc.at[pl.ds(sid * ROWS_PER_SUB, ROWS_PER_SUB)])
    plsc.subcore_barrier()

    def fire(j, b, sem):
        base = base_w + j * C
        pltpu.async_copy(src_hbm.at[pl.ds(base, C)], idx_v.at[b], sem)
        pltpu.async_copy(z_hbm.at[pl.ds(base, C)], rows_v.at[b], sem)

    def drain_add(j, b, sem):
        base = base_w + j * C
        pltpu.make_async_copy(src_hbm.at[pl.ds(base, C)], idx_v.at[b], sem).wait()
        pltpu.make_async_copy(z_hbm.at[pl.ds(base, C)], rows_v.at[b], sem).wait()
        pltpu.sync_copy(rows_v.at[b], acc.at[idx_v.at[b]], add=True)

    fire(0, 0, sem0)

    def g_loop(g, _):
        fire(2 * g + 1, 1, sem1)
        drain_add(2 * g, 0, sem0)

        @pl.when(2 * g + 2 < NFULL)
        def _():
            fire(2 * g + 2, 0, sem0)

        drain_add(2 * g + 1, 1, sem1)
        return 0

    lax.fori_loop(0, NFULL // 2, g_loop, 0)

    base = base_w + NFULL * C
    pltpu.sync_copy(src_hbm.at[pl.ds(base, TAIL)], idx_v3)
    pltpu.sync_copy(z_hbm.at[pl.ds(base, TAIL)], rows_v3)
    pltpu.sync_copy(rows_v3, acc.at[idx_v3], add=True)

    plsc.subcore_barrier()
    pltpu.sync_copy(acc.at[pl.ds(sid * ROWS_PER_SUB, ROWS_PER_SUB)],
                    part_hbm.at[cid].at[pl.ds(sid * ROWS_PER_SUB, ROWS_PER_SUB)])


BE = 2000


def _edge_mlp_body(rij_ref, inv_ref, cut_ref, cs_ref, pw_ref, ni_ref, nj_ref,
                   w1_ref, w2_ref, w2gh_ref, wgh_ref, wm_ref, z_ref):
    ni = ni_ref[...].astype(jnp.bfloat16)
    nj = nj_ref[...].astype(jnp.bfloat16)
    r = rij_ref[...]
    rb = inv_ref[...].astype(jnp.bfloat16)
    delta = (ni - nj) * rb
    fe = jnp.concatenate([ni, nj, delta], axis=1)
    g = jnp.dot(fe, wgh_ref[...], preferred_element_type=jnp.float32)
    g = 0.5 * jnp.tanh(g) + 0.5
    m = jnp.dot(fe, wm_ref[...], preferred_element_type=jnp.float32)
    m = jnp.where(m > 0, m, jnp.exp(jnp.minimum(m, 0.0)) - 1.0)
    pw = pw_ref[...]
    gate = jnp.dot(pw.astype(jnp.bfloat16), w2gh_ref[...],
                   preferred_element_type=jnp.float32)
    gate = 0.5 * jnp.tanh(gate) + 0.5
    z2 = jnp.dot((pw * gate).astype(jnp.bfloat16), w2_ref[...],
                 preferred_element_type=jnp.float32)
    z1 = jnp.dot(cs_ref[...].astype(jnp.bfloat16), w1_ref[...],
                 preferred_element_type=jnp.float32)
    mask = (r < cut_ref[0]).astype(jnp.float32)
    z_ref[...] = g * m * (z1 + z2) * mask


def _edge_mlp(rij2, inv2, cutoff, cs, pw, ni, nj, w1, w2, w2g, wg, wm):
    grid = (E // BE,)
    full = lambda shape: pl.BlockSpec(shape, lambda i: (0,) * len(shape))
    return pl.pallas_call(
        _edge_mlp_body,
        grid=grid,
        in_specs=[
            pl.BlockSpec((BE, 1), lambda i: (i, 0)),
            pl.BlockSpec((BE, 1), lambda i: (i, 0)),
            pl.BlockSpec(memory_space=pltpu.SMEM),
            pl.BlockSpec((BE, K), lambda i: (i, 0)),
            pl.BlockSpec((BE, K), lambda i: (i, 0)),
            pl.BlockSpec((BE, D), lambda i: (i, 0)),
            pl.BlockSpec((BE, D), lambda i: (i, 0)),
            full((K, D)),
            full((K, D)),
            full((K, K)),
            full((INF, D)),
            full((INF, D)),
        ],
        out_specs=pl.BlockSpec((BE, D), lambda i: (i, 0)),
        out_shape=jax.ShapeDtypeStruct((E, D), jnp.float32),
    )(rij2, inv2, cutoff, cs, pw, ni, nj, w1, w2, w2g, wg, wm)


BN = 1000


def _combine_body(x_ref, p_ref, o_ref):
    o_ref[...] = x_ref[...] + p_ref[0] + p_ref[1]


def _combine(x, parts):
    return pl.pallas_call(
        _combine_body,
        grid=(N // BN,),
        in_specs=[
            pl.BlockSpec((BN, D), lambda i: (i, 0)),
            pl.BlockSpec((NC, BN, D), lambda i: (0, i, 0)),
        ],
        out_specs=pl.BlockSpec((BN, D), lambda i: (i, 0)),
        out_shape=jax.ShapeDtypeStruct((N, D), jnp.float32),
    )(x, parts)


def kernel(input, nodes, edge_sources, edge_targets, rij, combine_sets,
           plane_wave, cutoff, W1, W2, W2g, Wg, Wm):
    bf = jnp.bfloat16
    ni, nj = _gather(input, edge_sources, edge_targets)
    z = _edge_mlp(rij[:, None], (1.0 / rij)[:, None], cutoff,
                  combine_sets, plane_wave, ni, nj,
                  W1.astype(bf), W2.astype(bf), (0.5 * W2g).astype(bf),
                  (0.5 * Wg).astype(bf), Wm.astype(bf))
    zero = jnp.zeros((N_ACC, D), jnp.float32)
    parts = _scatter(z, edge_sources, zero)
    return _combine(input, parts)


# speedup vs baseline: 2.3808x; 1.0733x over previous
import functools

import jax
import jax.numpy as jnp
from jax import lax
from jax.experimental import pallas as pl
from jax.experimental.pallas import tpu as pltpu
from jax.experimental.pallas import tpu_sc as plsc

N = 10000
E = 320000
D = 128
K = 64
INF = 3 * D

NC = 2
NS = 16
NW = NC * NS
NSEG = 5
ESEG = E // NSEG
PER_W = ESEG // NW
C = 128
NFULL = PER_W // C
TAIL = PER_W - NFULL * C
N_ACC = 10240
ROWS_PER_SUB = N_ACC // NS

_sc_mesh = plsc.VectorSubcoreMesh(core_axis_name="c", subcore_axis_name="s")


def _worker_id():
    return lax.axis_index("s") * NC + lax.axis_index("c")


@functools.lru_cache(maxsize=None)
def _make_gather(seg):
    seg_off = seg * ESEG

    @functools.partial(
        pl.kernel,
        out_type=(
            jax.ShapeDtypeStruct((ESEG, D), jnp.float32),
            jax.ShapeDtypeStruct((ESEG, D), jnp.float32),
        ),
        mesh=_sc_mesh,
        scratch_types=(
            pltpu.VMEM((3, C), jnp.int32),
            pltpu.VMEM((3, C), jnp.int32),
            pltpu.VMEM((3, C, D), jnp.float32),
            pltpu.VMEM((3, C, D), jnp.float32),
            pltpu.SemaphoreType.DMA,
            pltpu.SemaphoreType.DMA,
            pltpu.SemaphoreType.DMA,
            pltpu.SemaphoreType.DMA,
            pltpu.SemaphoreType.DMA,
            pltpu.SemaphoreType.DMA,
            pltpu.SemaphoreType.DMA,
            pltpu.SemaphoreType.DMA,
            pltpu.SemaphoreType.DMA,
            pltpu.VMEM((TAIL,), jnp.int32),
            pltpu.VMEM((TAIL,), jnp.int32),
            pltpu.VMEM((TAIL, D), jnp.float32),
            pltpu.VMEM((TAIL, D), jnp.float32),
        ),
    )
    def _gather(x_hbm, src_hbm, tgt_hbm, ni_hbm, nj_hbm,
                idx_s, idx_t, rows_s, rows_t,
                semI0, semI1, semI2, semG0, semG1, semG2, semW0, semW1, semW2,
                idx_s3, idx_t3, rows_s3, rows_t3):
        semI = (semI0, semI1, semI2)
        semG = (semG0, semG1, semG2)
        semW = (semW0, semW1, semW2)
        wid = _worker_id()
        lbase_w = wid * PER_W
        gbase_w = seg_off + lbase_w

        def fire_idx(j, b):
            gb = gbase_w + j * C
            pltpu.async_copy(src_hbm.at[pl.ds(gb, C)], idx_s.at[b], semI[b])
            pltpu.async_copy(tgt_hbm.at[pl.ds(gb, C)], idx_t.at[b], semI[b])

        def fire_gather(j, b):
            gb = gbase_w + j * C
            pltpu.make_async_copy(src_hbm.at[pl.ds(gb, C)], idx_s.at[b],
                                  semI[b]).wait()
            pltpu.make_async_copy(tgt_hbm.at[pl.ds(gb, C)], idx_t.at[b],
                                  semI[b]).wait()
            pltpu.async_copy(x_hbm.at[idx_s.at[b]], rows_s.at[b], semG[b])
            pltpu.async_copy(x_hbm.at[idx_t.at[b]], rows_t.at[b], semG[b])

        def fire_write(j, b):
            lb = lbase_w + j * C
            pltpu.make_async_copy(x_hbm.at[idx_s.at[b]], rows_s.at[b],
                                  semG[b]).wait()
            pltpu.make_async_copy(x_hbm.at[idx_t.at[b]], rows_t.at[b],
                                  semG[b]).wait()
            pltpu.async_copy(rows_s.at[b], ni_hbm.at[pl.ds(lb, C)], semW[b])
            pltpu.async_copy(rows_t.at[b], nj_hbm.at[pl.ds(lb, C)], semW[b])

        def wait_write(j, b):
            lb = lbase_w + j * C
            pltpu.make_async_copy(rows_s.at[b], ni_hbm.at[pl.ds(lb, C)],
                                  semW[b]).wait()
            pltpu.make_async_copy(rows_t.at[b], nj_hbm.at[pl.ds(lb, C)],
                                  semW[b]).wait()

        fire_idx(0, 0)
        fire_idx(1, 1)
        fire_gather(0, 0)

        def g_loop(g, _):
            for t in range(3):
                j = 3 * g + t
                bA = (t + 2) % 3

                if t == 0:
                    @pl.when(g > 0)
                    def _():
                        wait_write(j - 1, bA)
                    fire_idx(j + 2, bA)
                else:
                    wait_write(j - 1, bA)

                    @pl.when(g < (NFULL // 3) - 1)
                    def _():
                        fire_idx(j + 2, bA)

                if t < 2:
                    fire_gather(j + 1, (t + 1) % 3)
                else:
                    @pl.when(g < (NFULL // 3) - 1)
                    def _():
                        fire_gather(j + 1, (t + 1) % 3)

                fire_write(j, t)
            return 0

        lax.fori_loop(0, NFULL // 3, g_loop, 0)
        wait_write(NFULL - 1, (NFULL - 1) % 3)

        gb = gbase_w + NFULL * C
        lb = lbase_w + NFULL * C
        pltpu.sync_copy(src_hbm.at[pl.ds(gb, TAIL)], idx_s3)
        pltpu.sync_copy(tgt_hbm.at[pl.ds(gb, TAIL)], idx_t3)
        a = pltpu.async_copy(x_hbm.at[idx_s3], rows_s3, semG0)
        b = pltpu.async_copy(x_hbm.at[idx_t3], rows_t3, semG1)
        a.wait()
        b.wait()
        pltpu.sync_copy(rows_s3, ni_hbm.at[pl.ds(lb, TAIL)])
        pltpu.sync_copy(rows_t3, nj_hbm.at[pl.ds(lb, TAIL)])

    return _gather


@functools.partial(
    pl.kernel,
    out_type=jax.ShapeDtypeStruct((NC, N_ACC, D), jnp.float32),
    mesh=_sc_mesh,
    scratch_types=(
        pltpu.VMEM_SHARED((N_ACC, D), jnp.float32),
        pltpu.VMEM((2, C), jnp.int32),
        pltpu.VMEM((2, C, D), jnp.float32),
        pltpu.SemaphoreType.DMA,
        pltpu.SemaphoreType.DMA,
        pltpu.VMEM((TAIL,), jnp.int32),
        pltpu.VMEM((TAIL, D), jnp.float32),
    ),
)
def _scatter(z0, z1, z2, z3, z4, src_hbm, zero_hbm, part_hbm,
             acc, idx_v, rows_v, sem0, sem1, idx_v3, rows_v3):
    cid = lax.axis_index("c")
    sid = lax.axis_index("s")
    wid = _worker_id()
    zsegs = (z0, z1, z2, z3, z4)

    pltpu.sync_copy(zero_hbm.at[pl.ds(sid * ROWS_PER_SUB, ROWS_PER_SUB)],
                    acc.at[pl.ds(sid * ROWS_PER_SUB, ROWS_PER_SUB)])
    plsc.subcore_barrier()

    for seg in range(NSEG):
        z_hbm = zsegs[seg]
        lbase_w = wid * PER_W
        gbase_w = seg * ESEG + lbase_w

        def fire(j, b, sem):
            pltpu.async_copy(src_hbm.at[pl.ds(gbase_w + j * C, C)],
                             idx_v.at[b], sem)
            pltpu.async_copy(z_hbm.at[pl.ds(lbase_w + j * C, C)],
                             rows_v.at[b], sem)

        def drain_add(j, b, sem):
            pltpu.make_async_copy(src_hbm.at[pl.ds(gbase_w + j * C, C)],
                                  idx_v.at[b], sem).wait()
            pltpu.make_async_copy(z_hbm.at[pl.ds(lbase_w + j * C, C)],
                                  rows_v.at[b], sem).wait()
            pltpu.sync_copy(rows_v.at[b], acc.at[idx_v.at[b]], add=True)

        fire(0, 0, sem0)

        def g_loop(g, _):
            fire(2 * g + 1, 1, sem1)
            drain_add(2 * g, 0, sem0)

            @pl.when(2 * g + 2 < NFULL)
            def _():
                fire(2 * g + 2, 0, sem0)

            drain_add(2 * g + 1, 1, sem1)
            return 0

        lax.fori_loop(0, NFULL // 2, g_loop, 0)

        drain_add(NFULL - 1, 0, sem0)

        pltpu.sync_copy(src_hbm.at[pl.ds(gbase_w + NFULL * C, TAIL)], idx_v3)
        pltpu.sync_copy(z_hbm.at[pl.ds(lbase_w + NFULL * C, TAIL)], rows_v3)
        pltpu.sync_copy(rows_v3, acc.at[idx_v3], add=True)

    plsc.subcore_barrier()
    pltpu.sync_copy(acc.at[pl.ds(sid * ROWS_PER_SUB, ROWS_PER_SUB)],
                    part_hbm.at[cid].at[pl.ds(sid * ROWS_PER_SUB, ROWS_PER_SUB)])


BE = 2000
SEG_BLK = ESEG // BE


def _edge_mlp_body(rij_ref, inv_ref, cut_ref, cs_ref, pw_ref, ni_ref, nj_ref,
                   w1_ref, w2_ref, w2gh_ref, wgm_ref, z_ref):
    ni = ni_ref[...]
    nj = nj_ref[...]
    r = rij_ref[...]
    delta = (ni - nj) * inv_ref[...]
    fe = jnp.concatenate([ni, nj, delta], axis=1)
    gm = jnp.dot(fe, wgm_ref[...], preferred_element_type=jnp.float32)
    g = 1.0 / (1.0 + jnp.exp(-gm[:, :D]))
    m = gm[:, D:]
    m = jnp.where(m > 0, m, jnp.exp(jnp.minimum(m, 0.0)) - 1.0)
    pw = pw_ref[...]
    gate = jnp.dot(pw, w2gh_ref[...], preferred_element_type=jnp.float32)
    gate = 1.0 / (1.0 + jnp.exp(-gate))
    z2 = jnp.dot(pw * gate, w2_ref[...], preferred_element_type=jnp.float32)
    z1 = jnp.dot(cs_ref[...], w1_ref[...], preferred_element_type=jnp.float32)
    mask = (r < cut_ref[0]).astype(jnp.float32)
    z_ref[...] = g * m * (z1 + z2) * mask


@functools.lru_cache(maxsize=None)
def _make_edge_mlp(seg):
    off = seg * SEG_BLK
    full = lambda shape: pl.BlockSpec(shape, lambda i: (0,) * len(shape))
    gmap = lambda i: (off + i, 0)
    lmap = lambda i: (i, 0)

    def run(rij2, inv2, cutoff, cs, pw, ni, nj, w1, w2, w2g, wgm):
        return pl.pallas_call(
            _edge_mlp_body,
            grid=(SEG_BLK,),
            in_specs=[
                pl.BlockSpec((BE, 1), gmap),
                pl.BlockSpec((BE, 1), gmap),
                pl.BlockSpec(memory_space=pltpu.SMEM),
                pl.BlockSpec((BE, K), gmap),
                pl.BlockSpec((BE, K), gmap),
                pl.BlockSpec((BE, D), lmap),
                pl.BlockSpec((BE, D), lmap),
                full((K, D)),
                full((K, D)),
                full((K, K)),
                full((INF, 2 * D)),
            ],
            out_specs=pl.BlockSpec((BE, D), lmap),
            out_shape=jax.ShapeDtypeStruct((ESEG, D), jnp.float32),
        )(rij2, inv2, cutoff, cs, pw, ni, nj, w1, w2, w2g, wgm)

    return run


BN = 1000


def _combine_body(x_ref, p_ref, o_ref):
    o_ref[...] = x_ref[...] + p_ref[0] + p_ref[1]


def _combine(x, parts):
    return pl.pallas_call(
        _combine_body,
        grid=(N // BN,),
        in_specs=[
            pl.BlockSpec((BN, D), lambda i: (i, 0)),
            pl.BlockSpec((NC, BN, D), lambda i: (0, i, 0)),
        ],
        out_specs=pl.BlockSpec((BN, D), lambda i: (i, 0)),
        out_shape=jax.ShapeDtypeStruct((N, D), jnp.float32),
    )(x, parts)


def kernel(input, nodes, edge_sources, edge_targets, rij, combine_sets,
           plane_wave, cutoff, W1, W2, W2g, Wg, Wm):
    rij2 = rij[:, None]
    inv2 = (1.0 / rij)[:, None]
    w1 = W1
    w2 = W2
    w2gh = W2g
    wgm = jnp.concatenate([Wg, Wm], axis=1)
    zs = []
    for seg in range(NSEG):
        ni, nj = _make_gather(seg)(input, edge_sources, edge_targets)
        zs.append(_make_edge_mlp(seg)(rij2, inv2, cutoff, combine_sets,
                                      plane_wave, ni, nj,
                                      w1, w2, w2gh, wgm))
    zero = jnp.zeros((N_ACC, D), jnp.float32)
    parts = _scatter(*zs, edge_sources, zero)
    return _combine(input, parts)


# speedup vs baseline: 2.4228x; 1.0176x over previous
import functools

import jax
import jax.numpy as jnp
from jax import lax
from jax.experimental import pallas as pl
from jax.experimental.pallas import tpu as pltpu
from jax.experimental.pallas import tpu_sc as plsc

N = 10000
E = 320000
D = 128
K = 64
INF = 3 * D

NC = 2
NS = 16
NW = NC * NS
NSEG = 1
ESEG = E // NSEG
PER_W = ESEG // NW
C = 128
NFULL = PER_W // C
TAIL = PER_W - NFULL * C
N_ACC = 10240
ROWS_PER_SUB = N_ACC // NS

_sc_mesh = plsc.VectorSubcoreMesh(core_axis_name="c", subcore_axis_name="s")


def _worker_id():
    return lax.axis_index("s") * NC + lax.axis_index("c")


@functools.lru_cache(maxsize=None)
def _make_gather(seg):
    seg_off = seg * ESEG

    @functools.partial(
        pl.kernel,
        out_type=(
            jax.ShapeDtypeStruct((ESEG, D), jnp.float32),
            jax.ShapeDtypeStruct((ESEG, D), jnp.float32),
        ),
        mesh=_sc_mesh,
        scratch_types=(
            pltpu.VMEM((3, C), jnp.int32),
            pltpu.VMEM((3, C), jnp.int32),
            pltpu.VMEM((3, C, D), jnp.float32),
            pltpu.VMEM((3, C, D), jnp.float32),
            pltpu.SemaphoreType.DMA,
            pltpu.SemaphoreType.DMA,
            pltpu.SemaphoreType.DMA,
            pltpu.SemaphoreType.DMA,
            pltpu.SemaphoreType.DMA,
            pltpu.SemaphoreType.DMA,
            pltpu.SemaphoreType.DMA,
            pltpu.SemaphoreType.DMA,
            pltpu.SemaphoreType.DMA,
            pltpu.VMEM((TAIL,), jnp.int32),
            pltpu.VMEM((TAIL,), jnp.int32),
            pltpu.VMEM((TAIL, D), jnp.float32),
            pltpu.VMEM((TAIL, D), jnp.float32),
        ),
    )
    def _gather(x_hbm, src_hbm, tgt_hbm, ni_hbm, nj_hbm,
                idx_s, idx_t, rows_s, rows_t,
                semI0, semI1, semI2, semG0, semG1, semG2, semW0, semW1, semW2,
                idx_s3, idx_t3, rows_s3, rows_t3):
        semI = (semI0, semI1, semI2)
        semG = (semG0, semG1, semG2)
        semW = (semW0, semW1, semW2)
        wid = _worker_id()
        lbase_w = wid * PER_W
        gbase_w = seg_off + lbase_w

        def fire_idx(j, b):
            gb = gbase_w + j * C
            pltpu.async_copy(src_hbm.at[pl.ds(gb, C)], idx_s.at[b], semI[b])
            pltpu.async_copy(tgt_hbm.at[pl.ds(gb, C)], idx_t.at[b], semI[b])

        def fire_gather(j, b):
            gb = gbase_w + j * C
            pltpu.make_async_copy(src_hbm.at[pl.ds(gb, C)], idx_s.at[b],
                                  semI[b]).wait()
            pltpu.make_async_copy(tgt_hbm.at[pl.ds(gb, C)], idx_t.at[b],
                                  semI[b]).wait()
            pltpu.async_copy(x_hbm.at[idx_s.at[b]], rows_s.at[b], semG[b])
            pltpu.async_copy(x_hbm.at[idx_t.at[b]], rows_t.at[b], semG[b])

        def fire_write(j, b):
            lb = lbase_w + j * C
            pltpu.make_async_copy(x_hbm.at[idx_s.at[b]], rows_s.at[b],
                                  semG[b]).wait()
            pltpu.make_async_copy(x_hbm.at[idx_t.at[b]], rows_t.at[b],
                                  semG[b]).wait()
            pltpu.async_copy(rows_s.at[b], ni_hbm.at[pl.ds(lb, C)], semW[b])
            pltpu.async_copy(rows_t.at[b], nj_hbm.at[pl.ds(lb, C)], semW[b])

        def wait_write(j, b):
            lb = lbase_w + j * C
            pltpu.make_async_copy(rows_s.at[b], ni_hbm.at[pl.ds(lb, C)],
                                  semW[b]).wait()
            pltpu.make_async_copy(rows_t.at[b], nj_hbm.at[pl.ds(lb, C)],
                                  semW[b]).wait()

        fire_idx(0, 0)
        fire_idx(1, 1)
        fire_gather(0, 0)

        def g_loop(g, _):
            for t in range(3):
                j = 3 * g + t
                bA = (t + 2) % 3

                if t == 0:
                    @pl.when(g > 0)
                    def _():
                        wait_write(j - 1, bA)
                    fire_idx(j + 2, bA)
                else:
                    wait_write(j - 1, bA)

                    @pl.when(g < (NFULL // 3) - 1)
                    def _():
                        fire_idx(j + 2, bA)

                if t < 2:
                    fire_gather(j + 1, (t + 1) % 3)
                else:
                    @pl.when(g < (NFULL // 3) - 1)
                    def _():
                        fire_gather(j + 1, (t + 1) % 3)

                fire_write(j, t)
            return 0

        lax.fori_loop(0, NFULL // 3, g_loop, 0)
        wait_write(NFULL - 1, (NFULL - 1) % 3)

        gb = gbase_w + NFULL * C
        lb = lbase_w + NFULL * C
        pltpu.sync_copy(src_hbm.at[pl.ds(gb, TAIL)], idx_s3)
        pltpu.sync_copy(tgt_hbm.at[pl.ds(gb, TAIL)], idx_t3)
        a = pltpu.async_copy(x_hbm.at[idx_s3], rows_s3, semG0)
        b = pltpu.async_copy(x_hbm.at[idx_t3], rows_t3, semG1)
        a.wait()
        b.wait()
        pltpu.sync_copy(rows_s3, ni_hbm.at[pl.ds(lb, TAIL)])
        pltpu.sync_copy(rows_t3, nj_hbm.at[pl.ds(lb, TAIL)])

    return _gather


@functools.partial(
    pl.kernel,
    out_type=jax.ShapeDtypeStruct((NC, N_ACC, D), jnp.float32),
    mesh=_sc_mesh,
    scratch_types=(
        pltpu.VMEM_SHARED((N_ACC, D), jnp.float32),
        pltpu.VMEM((2, C), jnp.int32),
        pltpu.VMEM((2, C, D), jnp.float32),
        pltpu.SemaphoreType.DMA,
        pltpu.SemaphoreType.DMA,
        pltpu.VMEM((TAIL,), jnp.int32),
        pltpu.VMEM((TAIL, D), jnp.float32),
    ),
)
def _scatter(z0, src_hbm, zero_hbm, part_hbm,
             acc, idx_v, rows_v, sem0, sem1, idx_v3, rows_v3):
    cid = lax.axis_index("c")
    sid = lax.axis_index("s")
    wid = _worker_id()
    zsegs = (z0,)

    pltpu.sync_copy(zero_hbm.at[pl.ds(sid * ROWS_PER_SUB, ROWS_PER_SUB)],
                    acc.at[pl.ds(sid * ROWS_PER_SUB, ROWS_PER_SUB)])
    plsc.subcore_barrier()

    for seg in range(NSEG):
        z_hbm = zsegs[seg]
        lbase_w = wid * PER_W
        gbase_w = seg * ESEG + lbase_w

        def fire(j, b, sem):
            pltpu.async_copy(src_hbm.at[pl.ds(gbase_w + j * C, C)],
                             idx_v.at[b], sem)
            pltpu.async_copy(z_hbm.at[pl.ds(lbase_w + j * C, C)],
                             rows_v.at[b], sem)

        def drain_add(j, b, sem):
            pltpu.make_async_copy(src_hbm.at[pl.ds(gbase_w + j * C, C)],
                                  idx_v.at[b], sem).wait()
            pltpu.make_async_copy(z_hbm.at[pl.ds(lbase_w + j * C, C)],
                                  rows_v.at[b], sem).wait()
            pltpu.sync_copy(rows_v.at[b], acc.at[idx_v.at[b]], add=True)

        fire(0, 0, sem0)

        def g_loop(g, _):
            fire(2 * g + 1, 1, sem1)
            drain_add(2 * g, 0, sem0)

            @pl.when(2 * g + 2 < NFULL)
            def _():
                fire(2 * g + 2, 0, sem0)

            drain_add(2 * g + 1, 1, sem1)
            return 0

        lax.fori_loop(0, NFULL // 2, g_loop, 0)

        if NFULL % 2 == 1:
            drain_add(NFULL - 1, 0, sem0)

        pltpu.sync_copy(src_hbm.at[pl.ds(gbase_w + NFULL * C, TAIL)], idx_v3)
        pltpu.sync_copy(z_hbm.at[pl.ds(lbase_w + NFULL * C, TAIL)], rows_v3)
        pltpu.sync_copy(rows_v3, acc.at[idx_v3], add=True)

    plsc.subcore_barrier()
    pltpu.sync_copy(acc.at[pl.ds(sid * ROWS_PER_SUB, ROWS_PER_SUB)],
                    part_hbm.at[cid].at[pl.ds(sid * ROWS_PER_SUB, ROWS_PER_SUB)])


BE = 2000
SEG_BLK = ESEG // BE


def _edge_mlp_body(rij_ref, inv_ref, cut_ref, cs_ref, pw_ref, ni_ref, nj_ref,
                   w1_ref, w2_ref, w2gh_ref, wgm_ref, z_ref):
    ni = ni_ref[...]
    nj = nj_ref[...]
    r = rij_ref[...]
    delta = (ni - nj) * inv_ref[...]
    fe = jnp.concatenate([ni, nj, delta], axis=1)
    gm = jnp.dot(fe, wgm_ref[...], preferred_element_type=jnp.float32)
    g = 1.0 / (1.0 + jnp.exp(-gm[:, :D]))
    m = gm[:, D:]
    m = jnp.where(m > 0, m, jnp.exp(jnp.minimum(m, 0.0)) - 1.0)
    pw = pw_ref[...]
    gate = jnp.dot(pw, w2gh_ref[...], preferred_element_type=jnp.float32)
    gate = 1.0 / (1.0 + jnp.exp(-gate))
    z2 = jnp.dot(pw * gate, w2_ref[...], preferred_element_type=jnp.float32)
    z1 = jnp.dot(cs_ref[...], w1_ref[...], preferred_element_type=jnp.float32)
    mask = (r < cut_ref[0]).astype(jnp.float32)
    z_ref[...] = g * m * (z1 + z2) * mask


@functools.lru_cache(maxsize=None)
def _make_edge_mlp(seg):
    off = seg * SEG_BLK
    full = lambda shape: pl.BlockSpec(shape, lambda i: (0,) * len(shape))
    gmap = lambda i: (off + i, 0)
    lmap = lambda i: (i, 0)

    def run(rij2, inv2, cutoff, cs, pw, ni, nj, w1, w2, w2g, wgm):
        return pl.pallas_call(
            _edge_mlp_body,
            grid=(SEG_BLK,),
            in_specs=[
                pl.BlockSpec((BE, 1), gmap),
                pl.BlockSpec((BE, 1), gmap),
                pl.BlockSpec(memory_space=pltpu.SMEM),
                pl.BlockSpec((BE, K), gmap),
                pl.BlockSpec((BE, K), gmap),
                pl.BlockSpec((BE, D), lmap),
                pl.BlockSpec((BE, D), lmap),
                full((K, D)),
                full((K, D)),
                full((K, K)),
                full((INF, 2 * D)),
            ],
            out_specs=pl.BlockSpec((BE, D), lmap),
            out_shape=jax.ShapeDtypeStruct((ESEG, D), jnp.float32),
        )(rij2, inv2, cutoff, cs, pw, ni, nj, w1, w2, w2g, wgm)

    return run


BN = 1000


def _combine_body(x_ref, p_ref, o_ref):
    o_ref[...] = x_ref[...] + p_ref[0] + p_ref[1]


def _combine(x, parts):
    return pl.pallas_call(
        _combine_body,
        grid=(N // BN,),
        in_specs=[
            pl.BlockSpec((BN, D), lambda i: (i, 0)),
            pl.BlockSpec((NC, BN, D), lambda i: (0, i, 0)),
        ],
        out_specs=pl.BlockSpec((BN, D), lambda i: (i, 0)),
        out_shape=jax.ShapeDtypeStruct((N, D), jnp.float32),
    )(x, parts)


def kernel(input, nodes, edge_sources, edge_targets, rij, combine_sets,
           plane_wave, cutoff, W1, W2, W2g, Wg, Wm):
    rij2 = rij[:, None]
    inv2 = (1.0 / rij)[:, None]
    w1 = W1
    w2 = W2
    w2gh = W2g
    wgm = jnp.concatenate([Wg, Wm], axis=1)
    zs = []
    for seg in range(NSEG):
        ni, nj = _make_gather(seg)(input, edge_sources, edge_targets)
        zs.append(_make_edge_mlp(seg)(rij2, inv2, cutoff, combine_sets,
                                      plane_wave, ni, nj,
                                      w1, w2, w2gh, wgm))
    zero = jnp.zeros((N_ACC, D), jnp.float32)
    parts = _scatter(*zs, edge_sources, zero)
    return _combine(input, parts)


# speedup vs baseline: 2.7179x; 1.1218x over previous
import functools

import jax
import jax.numpy as jnp
from jax import lax
from jax.experimental import pallas as pl
from jax.experimental.pallas import tpu as pltpu
from jax.experimental.pallas import tpu_sc as plsc

N = 10000
E = 320000
D = 128
K = 64
INF = 3 * D

NC = 2
NS = 16
NW = NC * NS
NSEG = 1
ESEG = E // NSEG
PER_W = ESEG // NW
C = 128
NFULL = PER_W // C
TAIL = PER_W - NFULL * C
N_ACC = 10240
ROWS_PER_SUB = N_ACC // NS

_sc_mesh = plsc.VectorSubcoreMesh(core_axis_name="c", subcore_axis_name="s")


def _worker_id():
    return lax.axis_index("s") * NC + lax.axis_index("c")


@functools.lru_cache(maxsize=None)
def _make_gather(seg):
    seg_off = seg * ESEG

    @functools.partial(
        pl.kernel,
        out_type=(
            jax.ShapeDtypeStruct((ESEG, D), jnp.float32),
            jax.ShapeDtypeStruct((ESEG, D), jnp.float32),
        ),
        mesh=_sc_mesh,
        scratch_types=(
            pltpu.VMEM((3, C), jnp.int32),
            pltpu.VMEM((3, C), jnp.int32),
            pltpu.VMEM((3, C, D), jnp.float32),
            pltpu.VMEM((3, C, D), jnp.float32),
            pltpu.SemaphoreType.DMA,
            pltpu.SemaphoreType.DMA,
            pltpu.SemaphoreType.DMA,
            pltpu.SemaphoreType.DMA,
            pltpu.SemaphoreType.DMA,
            pltpu.SemaphoreType.DMA,
            pltpu.SemaphoreType.DMA,
            pltpu.SemaphoreType.DMA,
            pltpu.SemaphoreType.DMA,
            pltpu.VMEM((TAIL,), jnp.int32),
            pltpu.VMEM((TAIL,), jnp.int32),
            pltpu.VMEM((TAIL, D), jnp.float32),
            pltpu.VMEM((TAIL, D), jnp.float32),
        ),
    )
    def _gather(x_hbm, src_hbm, tgt_hbm, ni_hbm, nj_hbm,
                idx_s, idx_t, rows_s, rows_t,
                semI0, semI1, semI2, semG0, semG1, semG2, semW0, semW1, semW2,
                idx_s3, idx_t3, rows_s3, rows_t3):
        semI = (semI0, semI1, semI2)
        semG = (semG0, semG1, semG2)
        semW = (semW0, semW1, semW2)
        wid = _worker_id()
        lbase_w = wid * PER_W
        gbase_w = seg_off + lbase_w

        def fire_idx(j, b):
            gb = gbase_w + j * C
            pltpu.async_copy(src_hbm.at[pl.ds(gb, C)], idx_s.at[b], semI[b])
            pltpu.async_copy(tgt_hbm.at[pl.ds(gb, C)], idx_t.at[b], semI[b])

        def fire_gather(j, b):
            gb = gbase_w + j * C
            pltpu.make_async_copy(src_hbm.at[pl.ds(gb, C)], idx_s.at[b],
                                  semI[b]).wait()
            pltpu.make_async_copy(tgt_hbm.at[pl.ds(gb, C)], idx_t.at[b],
                                  semI[b]).wait()
            pltpu.async_copy(x_hbm.at[idx_s.at[b]], rows_s.at[b], semG[b])
            pltpu.async_copy(x_hbm.at[idx_t.at[b]], rows_t.at[b], semG[b])

        def fire_write(j, b):
            lb = lbase_w + j * C
            pltpu.make_async_copy(x_hbm.at[idx_s.at[b]], rows_s.at[b],
                                  semG[b]).wait()
            pltpu.make_async_copy(x_hbm.at[idx_t.at[b]], rows_t.at[b],
                                  semG[b]).wait()
            pltpu.async_copy(rows_s.at[b], ni_hbm.at[pl.ds(lb, C)], semW[b])
            pltpu.async_copy(rows_t.at[b], nj_hbm.at[pl.ds(lb, C)], semW[b])

        def wait_write(j, b):
            lb = lbase_w + j * C
            pltpu.make_async_copy(rows_s.at[b], ni_hbm.at[pl.ds(lb, C)],
                                  semW[b]).wait()
            pltpu.make_async_copy(rows_t.at[b], nj_hbm.at[pl.ds(lb, C)],
                                  semW[b]).wait()

        fire_idx(0, 0)
        fire_idx(1, 1)
        fire_gather(0, 0)

        def g_loop(g, _):
            for t in range(3):
                j = 3 * g + t
                bA = (t + 2) % 3

                if t == 0:
                    @pl.when(g > 0)
                    def _():
                        wait_write(j - 1, bA)
                    fire_idx(j + 2, bA)
                else:
                    wait_write(j - 1, bA)

                    @pl.when(g < (NFULL // 3) - 1)
                    def _():
                        fire_idx(j + 2, bA)

                if t < 2:
                    fire_gather(j + 1, (t + 1) % 3)
                else:
                    @pl.when(g < (NFULL // 3) - 1)
                    def _():
                        fire_gather(j + 1, (t + 1) % 3)

                fire_write(j, t)
            return 0

        lax.fori_loop(0, NFULL // 3, g_loop, 0)
        wait_write(NFULL - 1, (NFULL - 1) % 3)

        gb = gbase_w + NFULL * C
        lb = lbase_w + NFULL * C
        pltpu.sync_copy(src_hbm.at[pl.ds(gb, TAIL)], idx_s3)
        pltpu.sync_copy(tgt_hbm.at[pl.ds(gb, TAIL)], idx_t3)
        a = pltpu.async_copy(x_hbm.at[idx_s3], rows_s3, semG0)
        b = pltpu.async_copy(x_hbm.at[idx_t3], rows_t3, semG1)
        a.wait()
        b.wait()
        pltpu.sync_copy(rows_s3, ni_hbm.at[pl.ds(lb, TAIL)])
        pltpu.sync_copy(rows_t3, nj_hbm.at[pl.ds(lb, TAIL)])

    return _gather


@functools.partial(
    pl.kernel,
    out_type=jax.ShapeDtypeStruct((NC, N_ACC, D), jnp.float32),
    mesh=_sc_mesh,
    scratch_types=(
        pltpu.VMEM_SHARED((N_ACC, D), jnp.float32),
        pltpu.VMEM((2, C), jnp.int32),
        pltpu.VMEM((2, C, D), jnp.float32),
        pltpu.SemaphoreType.DMA,
        pltpu.SemaphoreType.DMA,
        pltpu.VMEM((TAIL,), jnp.int32),
        pltpu.VMEM((TAIL, D), jnp.float32),
    ),
)
def _scatter(z0, src_hbm, zero_hbm, part_hbm,
             acc, idx_v, rows_v, sem0, sem1, idx_v3, rows_v3):
    cid = lax.axis_index("c")
    sid = lax.axis_index("s")
    wid = _worker_id()
    zsegs = (z0,)

    pltpu.sync_copy(zero_hbm.at[pl.ds(sid * ROWS_PER_SUB, ROWS_PER_SUB)],
                    acc.at[pl.ds(sid * ROWS_PER_SUB, ROWS_PER_SUB)])
    plsc.subcore_barrier()

    for seg in range(NSEG):
        z_hbm = zsegs[seg]
        lbase_w = wid * PER_W
        gbase_w = seg * ESEG + lbase_w

        def fire(j, b, sem):
            pltpu.async_copy(src_hbm.at[pl.ds(gbase_w + j * C, C)],
                             idx_v.at[b], sem)
            pltpu.async_copy(z_hbm.at[pl.ds(lbase_w + j * C, C)],
                             rows_v.at[b], sem)

        def drain_add(j, b, sem):
            pltpu.make_async_copy(src_hbm.at[pl.ds(gbase_w + j * C, C)],
                                  idx_v.at[b], sem).wait()
            pltpu.make_async_copy(z_hbm.at[pl.ds(lbase_w + j * C, C)],
                                  rows_v.at[b], sem).wait()
            pltpu.sync_copy(rows_v.at[b], acc.at[idx_v.at[b]], add=True)

        fire(0, 0, sem0)

        def g_loop(g, _):
            fire(2 * g + 1, 1, sem1)
            drain_add(2 * g, 0, sem0)

            @pl.when(2 * g + 2 < NFULL)
            def _():
                fire(2 * g + 2, 0, sem0)

            drain_add(2 * g + 1, 1, sem1)
            return 0

        lax.fori_loop(0, NFULL // 2, g_loop, 0)

        if NFULL % 2 == 1:
            drain_add(NFULL - 1, 0, sem0)

        pltpu.sync_copy(src_hbm.at[pl.ds(gbase_w + NFULL * C, TAIL)], idx_v3)
        pltpu.sync_copy(z_hbm.at[pl.ds(lbase_w + NFULL * C, TAIL)], rows_v3)
        pltpu.sync_copy(rows_v3, acc.at[idx_v3], add=True)

    plsc.subcore_barrier()
    pltpu.sync_copy(acc.at[pl.ds(sid * ROWS_PER_SUB, ROWS_PER_SUB)],
                    part_hbm.at[cid].at[pl.ds(sid * ROWS_PER_SUB, ROWS_PER_SUB)])


BE = 2000
SEG_BLK = ESEG // BE


def _edge_mlp_body(inv_ref, icut_ref, cs_ref, pw_ref, ni_ref, nj_ref,
                   w1_ref, w2_ref, w2gh_ref, wgm_ref, z_ref):
    ni = ni_ref[...]
    nj = nj_ref[...]
    inv = inv_ref[...]
    delta = (ni - nj) * inv
    fe = jnp.concatenate([ni, nj, delta], axis=1).astype(jnp.bfloat16)
    gm = jnp.dot(fe, wgm_ref[...], preferred_element_type=jnp.float32)
    g = 1.0 / (1.0 + jnp.exp(-gm[:, :D]))
    m = gm[:, D:]
    m = jnp.where(m > 0, m, jnp.exp(jnp.minimum(m, 0.0)) - 1.0)
    pw = pw_ref[...]
    gate = jnp.dot(pw, w2gh_ref[...], preferred_element_type=jnp.float32)
    gate = 1.0 / (1.0 + jnp.exp(-gate))
    z2 = jnp.dot(pw * gate, w2_ref[...], preferred_element_type=jnp.float32)
    z1 = jnp.dot(cs_ref[...], w1_ref[...], preferred_element_type=jnp.float32)
    mask = (inv > icut_ref[0]).astype(jnp.float32)
    z_ref[...] = g * m * (z1 + z2) * mask


@functools.lru_cache(maxsize=None)
def _make_edge_mlp(seg):
    off = seg * SEG_BLK
    full = lambda shape: pl.BlockSpec(shape, lambda i: (0,) * len(shape))
    gmap = lambda i: (off + i, 0)
    lmap = lambda i: (i, 0)

    def run(inv2, icut, cs, pw, ni, nj, w1, w2, w2g, wgm):
        return pl.pallas_call(
            _edge_mlp_body,
            grid=(SEG_BLK,),
            in_specs=[
                pl.BlockSpec((BE, 1), gmap),
                pl.BlockSpec(memory_space=pltpu.SMEM),
                pl.BlockSpec((BE, K), gmap),
                pl.BlockSpec((BE, K), gmap),
                pl.BlockSpec((BE, D), lmap),
                pl.BlockSpec((BE, D), lmap),
                full((K, D)),
                full((K, D)),
                full((K, K)),
                full((INF, 2 * D)),
            ],
            out_specs=pl.BlockSpec((BE, D), lmap),
            out_shape=jax.ShapeDtypeStruct((ESEG, D), jnp.float32),
        )(inv2, icut, cs, pw, ni, nj, w1, w2, w2g, wgm)

    return run


BN = 1000


def _combine_body(x_ref, p_ref, o_ref):
    o_ref[...] = x_ref[...] + p_ref[0] + p_ref[1]


def _combine(x, parts):
    return pl.pallas_call(
        _combine_body,
        grid=(N // BN,),
        in_specs=[
            pl.BlockSpec((BN, D), lambda i: (i, 0)),
            pl.BlockSpec((NC, BN, D), lambda i: (0, i, 0)),
        ],
        out_specs=pl.BlockSpec((BN, D), lambda i: (i, 0)),
        out_shape=jax.ShapeDtypeStruct((N, D), jnp.float32),
    )(x, parts)


def kernel(input, nodes, edge_sources, edge_targets, rij, combine_sets,
           plane_wave, cutoff, W1, W2, W2g, Wg, Wm):
    inv2 = (1.0 / rij)[:, None]
    icut = 1.0 / cutoff
    w1 = W1
    w2 = W2
    w2gh = W2g
    wgm = jnp.concatenate([Wg, Wm], axis=1).astype(jnp.bfloat16)
    zs = []
    for seg in range(NSEG):
        ni, nj = _make_gather(seg)(input, edge_sources, edge_targets)
        zs.append(_make_edge_mlp(seg)(inv2, icut, combine_sets,
                                      plane_wave, ni, nj,
                                      w1, w2, w2gh, wgm))
    zero = jnp.zeros((N_ACC, D), jnp.float32)
    parts = _scatter(*zs, edge_sources, zero)
    return _combine(input, parts)


# speedup vs baseline: 2.8527x; 1.0496x over previous
import functools

import jax
import jax.numpy as jnp
from jax import lax
from jax.experimental import pallas as pl
from jax.experimental.pallas import tpu as pltpu
from jax.experimental.pallas import tpu_sc as plsc

N = 10000
E = 320000
D = 128
K = 64
INF = 3 * D

NC = 2
NS = 16
NW = NC * NS
NSEG = 1
ESEG = E // NSEG
PER_W = ESEG // NW
C = 128
NFULL = PER_W // C
TAIL = PER_W - NFULL * C
N_ACC = 10240
ROWS_PER_SUB = N_ACC // NS

_sc_mesh = plsc.VectorSubcoreMesh(core_axis_name="c", subcore_axis_name="s")


def _worker_id():
    return lax.axis_index("s") * NC + lax.axis_index("c")


@functools.lru_cache(maxsize=None)
def _make_gather(seg):
    seg_off = seg * ESEG

    @functools.partial(
        pl.kernel,
        out_type=(
            jax.ShapeDtypeStruct((ESEG, D), jnp.float32),
            jax.ShapeDtypeStruct((ESEG, D), jnp.float32),
        ),
        mesh=_sc_mesh,
        scratch_types=(
            pltpu.VMEM((3, C), jnp.int32),
            pltpu.VMEM((3, C), jnp.int32),
            pltpu.VMEM((3, C, D), jnp.float32),
            pltpu.VMEM((3, C, D), jnp.float32),
            pltpu.SemaphoreType.DMA,
            pltpu.SemaphoreType.DMA,
            pltpu.SemaphoreType.DMA,
            pltpu.SemaphoreType.DMA,
            pltpu.SemaphoreType.DMA,
            pltpu.SemaphoreType.DMA,
            pltpu.SemaphoreType.DMA,
            pltpu.SemaphoreType.DMA,
            pltpu.SemaphoreType.DMA,
            pltpu.VMEM((TAIL,), jnp.int32),
            pltpu.VMEM((TAIL,), jnp.int32),
            pltpu.VMEM((TAIL, D), jnp.float32),
            pltpu.VMEM((TAIL, D), jnp.float32),
        ),
    )
    def _gather(x_hbm, src_hbm, tgt_hbm, ni_hbm, nj_hbm,
                idx_s, idx_t, rows_s, rows_t,
                semI0, semI1, semI2, semG0, semG1, semG2, semW0, semW1, semW2,
                idx_s3, idx_t3, rows_s3, rows_t3):
        semI = (semI0, semI1, semI2)
        semG = (semG0, semG1, semG2)
        semW = (semW0, semW1, semW2)
        wid = _worker_id()
        lbase_w = wid * PER_W
        gbase_w = seg_off + lbase_w

        def fire_idx(j, b):
            gb = gbase_w + j * C
            pltpu.async_copy(src_hbm.at[pl.ds(gb, C)], idx_s.at[b], semI[b])
            pltpu.async_copy(tgt_hbm.at[pl.ds(gb, C)], idx_t.at[b], semI[b])

        def fire_gather(j, b):
            gb = gbase_w + j * C
            pltpu.make_async_copy(src_hbm.at[pl.ds(gb, C)], idx_s.at[b],
                                  semI[b]).wait()
            pltpu.make_async_copy(tgt_hbm.at[pl.ds(gb, C)], idx_t.at[b],
                                  semI[b]).wait()
            pltpu.async_copy(x_hbm.at[idx_s.at[b]], rows_s.at[b], semG[b])
            pltpu.async_copy(x_hbm.at[idx_t.at[b]], rows_t.at[b], semG[b])

        def fire_write(j, b):
            lb = lbase_w + j * C
            pltpu.make_async_copy(x_hbm.at[idx_s.at[b]], rows_s.at[b],
                                  semG[b]).wait()
            pltpu.make_async_copy(x_hbm.at[idx_t.at[b]], rows_t.at[b],
                                  semG[b]).wait()
            pltpu.async_copy(rows_s.at[b], ni_hbm.at[pl.ds(lb, C)], semW[b])
            pltpu.async_copy(rows_t.at[b], nj_hbm.at[pl.ds(lb, C)], semW[b])

        def wait_write(j, b):
            lb = lbase_w + j * C
            pltpu.make_async_copy(rows_s.at[b], ni_hbm.at[pl.ds(lb, C)],
                                  semW[b]).wait()
            pltpu.make_async_copy(rows_t.at[b], nj_hbm.at[pl.ds(lb, C)],
                                  semW[b]).wait()

        fire_idx(0, 0)
        fire_idx(1, 1)
        fire_gather(0, 0)

        def g_loop(g, _):
            for t in range(3):
                j = 3 * g + t
                bA = (t + 2) % 3

                if t == 0:
                    @pl.when(g > 0)
                    def _():
                        wait_write(j - 1, bA)
                    fire_idx(j + 2, bA)
                else:
                    wait_write(j - 1, bA)

                    @pl.when(g < (NFULL // 3) - 1)
                    def _():
                        fire_idx(j + 2, bA)

                if t < 2:
                    fire_gather(j + 1, (t + 1) % 3)
                else:
                    @pl.when(g < (NFULL // 3) - 1)
                    def _():
                        fire_gather(j + 1, (t + 1) % 3)

                fire_write(j, t)
            return 0

        lax.fori_loop(0, NFULL // 3, g_loop, 0)
        wait_write(NFULL - 1, (NFULL - 1) % 3)

        gb = gbase_w + NFULL * C
        lb = lbase_w + NFULL * C
        pltpu.sync_copy(src_hbm.at[pl.ds(gb, TAIL)], idx_s3)
        pltpu.sync_copy(tgt_hbm.at[pl.ds(gb, TAIL)], idx_t3)
        a = pltpu.async_copy(x_hbm.at[idx_s3], rows_s3, semG0)
        b = pltpu.async_copy(x_hbm.at[idx_t3], rows_t3, semG1)
        a.wait()
        b.wait()
        pltpu.sync_copy(rows_s3, ni_hbm.at[pl.ds(lb, TAIL)])
        pltpu.sync_copy(rows_t3, nj_hbm.at[pl.ds(lb, TAIL)])

    return _gather


@functools.partial(
    pl.kernel,
    out_type=jax.ShapeDtypeStruct((NC, N_ACC, D), jnp.float32),
    mesh=_sc_mesh,
    scratch_types=(
        pltpu.VMEM_SHARED((N_ACC, D), jnp.float32),
        pltpu.VMEM((2, C), jnp.int32),
        pltpu.VMEM((2, C, D), jnp.float32),
        pltpu.SemaphoreType.DMA,
        pltpu.SemaphoreType.DMA,
        pltpu.VMEM((TAIL,), jnp.int32),
        pltpu.VMEM((TAIL, D), jnp.float32),
    ),
)
def _scatter(z0, src_hbm, zero_hbm, part_hbm,
             acc, idx_v, rows_v, sem0, sem1, idx_v3, rows_v3):
    cid = lax.axis_index("c")
    sid = lax.axis_index("s")
    wid = _worker_id()
    zsegs = (z0,)

    pltpu.sync_copy(zero_hbm.at[pl.ds(sid * ROWS_PER_SUB, ROWS_PER_SUB)],
                    acc.at[pl.ds(sid * ROWS_PER_SUB, ROWS_PER_SUB)])
    plsc.subcore_barrier()

    for seg in range(NSEG):
        z_hbm = zsegs[seg]
        lbase_w = wid * PER_W
        gbase_w = seg * ESEG + lbase_w

        def fire(j, b, sem):
            pltpu.async_copy(src_hbm.at[pl.ds(gbase_w + j * C, C)],
                             idx_v.at[b], sem)
            pltpu.async_copy(z_hbm.at[pl.ds(lbase_w + j * C, C)],
                             rows_v.at[b], sem)

        def drain_add(j, b, sem):
            pltpu.make_async_copy(src_hbm.at[pl.ds(gbase_w + j * C, C)],
                                  idx_v.at[b], sem).wait()
            pltpu.make_async_copy(z_hbm.at[pl.ds(lbase_w + j * C, C)],
                                  rows_v.at[b], sem).wait()
            pltpu.sync_copy(rows_v.at[b], acc.at[idx_v.at[b]], add=True)

        fire(0, 0, sem0)

        def g_loop(g, _):
            fire(2 * g + 1, 1, sem1)
            drain_add(2 * g, 0, sem0)

            @pl.when(2 * g + 2 < NFULL)
            def _():
                fire(2 * g + 2, 0, sem0)

            drain_add(2 * g + 1, 1, sem1)
            return 0

        lax.fori_loop(0, NFULL // 2, g_loop, 0)

        if NFULL % 2 == 1:
            drain_add(NFULL - 1, 0, sem0)

        pltpu.sync_copy(src_hbm.at[pl.ds(gbase_w + NFULL * C, TAIL)], idx_v3)
        pltpu.sync_copy(z_hbm.at[pl.ds(lbase_w + NFULL * C, TAIL)], rows_v3)
        pltpu.sync_copy(rows_v3, acc.at[idx_v3], add=True)

    plsc.subcore_barrier()
    pltpu.sync_copy(acc.at[pl.ds(sid * ROWS_PER_SUB, ROWS_PER_SUB)],
                    part_hbm.at[cid].at[pl.ds(sid * ROWS_PER_SUB, ROWS_PER_SUB)])


BE = 4000
SEG_BLK = ESEG // BE


def _edge_mlp_body(inv_ref, icut_ref, cs_ref, pw_ref, ni_ref, nj_ref,
                   w1_ref, w2_ref, w2gh_ref, wgm_ref, z_ref):
    ni = ni_ref[...]
    nj = nj_ref[...]
    inv = inv_ref[...]
    delta = (ni - nj) * inv
    fe = jnp.concatenate([ni, nj, delta], axis=1).astype(jnp.bfloat16)
    gm = jnp.dot(fe, wgm_ref[...], preferred_element_type=jnp.float32)
    g = 1.0 / (1.0 + jnp.exp(-gm[:, :D]))
    m = gm[:, D:]
    m = jnp.where(m > 0, m, jnp.exp(jnp.minimum(m, 0.0)) - 1.0)
    pw = pw_ref[...]
    gate = jnp.dot(pw, w2gh_ref[...], preferred_element_type=jnp.float32)
    gate = 1.0 / (1.0 + jnp.exp(-gate))
    z2 = jnp.dot(pw * gate, w2_ref[...], preferred_element_type=jnp.float32)
    z1 = jnp.dot(cs_ref[...], w1_ref[...], preferred_element_type=jnp.float32)
    mask = (inv > icut_ref[0]).astype(jnp.float32)
    z_ref[...] = g * m * (z1 + z2) * mask


@functools.lru_cache(maxsize=None)
def _make_edge_mlp(seg):
    off = seg * SEG_BLK
    full = lambda shape: pl.BlockSpec(shape, lambda i: (0,) * len(shape))
    gmap = lambda i: (off + i, 0)
    lmap = lambda i: (i, 0)

    def run(inv2, icut, cs, pw, ni, nj, w1, w2, w2g, wgm):
        return pl.pallas_call(
            _edge_mlp_body,
            grid=(SEG_BLK,),
            in_specs=[
                pl.BlockSpec((BE, 1), gmap),
                pl.BlockSpec(memory_space=pltpu.SMEM),
                pl.BlockSpec((BE, K), gmap),
                pl.BlockSpec((BE, K), gmap),
                pl.BlockSpec((BE, D), lmap),
                pl.BlockSpec((BE, D), lmap),
                full((K, D)),
                full((K, D)),
                full((K, K)),
                full((INF, 2 * D)),
            ],
            out_specs=pl.BlockSpec((BE, D), lmap),
            out_shape=jax.ShapeDtypeStruct((ESEG, D), jnp.float32),
        )(inv2, icut, cs, pw, ni, nj, w1, w2, w2g, wgm)

    return run


BN = 1000


def _combine_body(x_ref, p_ref, o_ref):
    o_ref[...] = x_ref[...] + p_ref[0] + p_ref[1]


def _combine(x, parts):
    return pl.pallas_call(
        _combine_body,
        grid=(N // BN,),
        in_specs=[
            pl.BlockSpec((BN, D), lambda i: (i, 0)),
            pl.BlockSpec((NC, BN, D), lambda i: (0, i, 0)),
        ],
        out_specs=pl.BlockSpec((BN, D), lambda i: (i, 0)),
        out_shape=jax.ShapeDtypeStruct((N, D), jnp.float32),
    )(x, parts)


def kernel(input, nodes, edge_sources, edge_targets, rij, combine_sets,
           plane_wave, cutoff, W1, W2, W2g, Wg, Wm):
    inv2 = (1.0 / rij)[:, None]
    icut = 1.0 / cutoff
    w1 = W1
    w2 = W2
    w2gh = W2g
    wgm = jnp.concatenate([Wg, Wm], axis=1).astype(jnp.bfloat16)
    zs = []
    for seg in range(NSEG):
        ni, nj = _make_gather(seg)(input, edge_sources, edge_targets)
        zs.append(_make_edge_mlp(seg)(inv2, icut, combine_sets,
                                      plane_wave, ni, nj,
                                      w1, w2, w2gh, wgm))
    zero = jnp.zeros((N_ACC, D), jnp.float32)
    parts = _scatter(*zs, edge_sources, zero)
    return _combine(input, parts)
